# Initial kernel scaffold; baseline (speedup 1.0000x reference)
import jax
import jax.numpy as jnp
from jax.experimental import pallas as pl


def kernel(x):
    raise NotImplementedError("write your pallas kernel here")


# speedup vs baseline: 12.5210x; 12.5210x over previous
import numpy as np
import jax
import jax.numpy as jnp
from jax.experimental import pallas as pl

_F = 2048
_L = 345
_R = 16
_C = 128
_NS = 128
_XF = 258
_HI = jax.lax.Precision.HIGHEST


def _build_consts():
    cols = []

    def add_region(npts, ncoord, base):
        r, c = np.triu_indices(npts, k=1)
        for cc in range(ncoord):
            m = np.zeros((_XF, len(r)), np.float32)
            m[base + 3 * r + cc, np.arange(len(r))] = 1.0
            m[base + 3 * c + cc, np.arange(len(r))] -= 1.0
            cols.append(m)

    add_region(21, 3, 0)
    add_region(25, 2, 63)
    add_region(20, 2, 138)
    add_region(20, 2, 198)
    dm = np.concatenate(cols, axis=1)

    sel = np.zeros((_XF, 153), np.float32)
    for i in range(63):
        sel[i, i] = 1.0
    for p in range(25):
        for cc in range(2):
            sel[63 + 3 * p + cc, 63 + 2 * p + cc] = 1.0
    for p in range(20):
        for cc in range(2):
            sel[138 + 3 * p + cc, 113 + 2 * p + cc] = 1.0
    return dm, sel


_DM_NP, _SEL_NP = _build_consts()


def _fgen_kernel(x_ref, dm_ref, sel_ref, out_ref):
    x = x_ref[...]
    x = jnp.where(jnp.isnan(x), jnp.float32(0.0), x)
    xr = x.reshape(_R, _C, _L)
    lh = xr[:, :, 120:183]
    rh = xr[:, :, 282:345]
    lcnt = jnp.sum((lh != 0.0).astype(jnp.float32))
    rcnt = jnp.sum((rh != 0.0).astype(jnp.float32))
    cond = lcnt > rcnt

    hand = jnp.where(cond, lh, rh)
    fs = jnp.sum(hand, axis=2)
    mask = (fs != 0.0).astype(jnp.float32)

    jj = jax.lax.broadcasted_iota(jnp.int32, (_C, _C), 0)
    kk = jax.lax.broadcasted_iota(jnp.int32, (_C, _C), 1)
    tri = (jj <= kk).astype(jnp.float32)
    crow = jnp.dot(mask, tri, precision=_HI)
    tot = crow[:, _C - 1:_C]
    aa = jax.lax.broadcasted_iota(jnp.int32, (_R, _R), 0)
    bb = jax.lax.broadcasted_iota(jnp.int32, (_R, _R), 1)
    lstrict = (bb < aa).astype(jnp.float32)
    pre = jnp.dot(lstrict, tot, precision=_HI)
    kinc = crow + pre
    nkept = pre[_R - 1:_R, 0:1] + tot[_R - 1:_R, 0:1]

    gr = jax.lax.broadcasted_iota(jnp.int32, (_R, _C), 0)
    gj = jax.lax.broadcasted_iota(jnp.int32, (_R, _C), 1)
    gidx = (gr * _C + gj).astype(jnp.float32)
    dinc = (gidx + 1.0) - kinc
    pos = jnp.where(mask > 0, kinc - 1.0, nkept + dinc - 1.0)

    p_iota = jax.lax.broadcasted_iota(jnp.int32, (_R, _C, _NS), 2)
    pos3 = pos.astype(jnp.int32)[:, :, None]
    oh = (pos3 == p_iota).astype(jnp.float32).reshape(_F, _NS)
    ohn = (pos3 == p_iota + 1).astype(jnp.float32).reshape(_F, _NS)
    g = jax.lax.dot_general(oh, x, (((0,), (0,)), ((), ())), precision=_HI)
    gn = jax.lax.dot_general(ohn, x, (((0,), (0,)), ((), ())), precision=_HI)

    def build_xf(gg):
        h = jnp.where(cond, gg[:, 120:183], gg[:, 282:345])
        xf = jnp.concatenate([h, gg[:, 183:258], gg[:, 0:120]], axis=1)
        lane = jax.lax.broadcasted_iota(jnp.int32, (_NS, _XF), 1)
        neg = jnp.logical_and(cond, lane % 3 == 0)
        return jnp.where(neg, -xf, xf)

    xf = build_xf(g)
    xfn = build_xf(gn)
    s_iota = jax.lax.broadcasted_iota(jnp.int32, (_NS, 1), 0).astype(jnp.float32)
    nk = (s_iota < (nkept - 1.0)).astype(jnp.float32)
    dxyz = (xf - xfn) * nk

    sel = sel_ref[...]
    coords = jnp.dot(xf, sel, precision=_HI)
    dcoords = jnp.dot(dxyz, sel, precision=_HI)
    dd = jnp.dot(xf, dm_ref[...], precision=_HI)
    hd = jnp.sqrt(dd[:, 0:210] ** 2 + dd[:, 210:420] ** 2 + dd[:, 420:630] ** 2)
    pd = jnp.sqrt(dd[:, 630:930] ** 2 + dd[:, 930:1230] ** 2)
    od = jnp.sqrt(dd[:, 1230:1420] ** 2 + dd[:, 1420:1610] ** 2)
    idd = jnp.sqrt(dd[:, 1610:1800] ** 2 + dd[:, 1800:1990] ** 2)

    out_ref[:, 0:153] = coords
    out_ref[:, 153:306] = dcoords
    out_ref[:, 306:516] = hd
    out_ref[:, 516:816] = pd
    out_ref[:, 816:1006] = od
    out_ref[:, 1006:1196] = idd


def kernel(x):
    x2 = x.reshape(_F, _L)
    dm = jnp.asarray(_DM_NP)
    sel = jnp.asarray(_SEL_NP)
    out = pl.pallas_call(
        _fgen_kernel,
        out_shape=jax.ShapeDtypeStruct((_NS, 1196), jnp.float32),
    )(x2, dm, sel)
    return out[:100].reshape(1, 100, 1196)


# speedup vs baseline: 13.6306x; 1.0886x over previous
import numpy as np
import jax
import jax.numpy as jnp
from jax.experimental import pallas as pl

_F = 2048
_L = 345
_R = 16
_C = 128
_NS = 128
_XF = 258
_HI = jax.lax.Precision.HIGHEST
_DF = jax.lax.Precision.DEFAULT


def _build_consts():
    cols = []
    sel = np.zeros((_XF, 153), np.float32)
    for i in range(63):
        sel[i, i] = 1.0
    for p in range(25):
        for cc in range(2):
            sel[63 + 3 * p + cc, 63 + 2 * p + cc] = 1.0
    for p in range(20):
        for cc in range(2):
            sel[138 + 3 * p + cc, 113 + 2 * p + cc] = 1.0
    cols.append(sel)

    def add_region(npts, ncoord, base):
        r, c = np.triu_indices(npts, k=1)
        for cc in range(ncoord):
            m = np.zeros((_XF, len(r)), np.float32)
            m[base + 3 * r + cc, np.arange(len(r))] = 1.0
            m[base + 3 * c + cc, np.arange(len(r))] -= 1.0
            cols.append(m)

    add_region(21, 3, 0)
    add_region(25, 2, 63)
    add_region(20, 2, 138)
    add_region(20, 2, 198)
    return np.concatenate(cols, axis=1)


_FM_NP = _build_consts()


def _fgen_kernel(x_ref, fm_ref, out_ref):
    x = x_ref[...]
    x = jnp.where(jnp.isnan(x), jnp.float32(0.0), x)
    xr = x.reshape(_R, _C, _L)
    lh = xr[:, :, 120:183]
    rh = xr[:, :, 282:345]
    lcnt = jnp.sum((lh != 0.0).astype(jnp.float32))
    rcnt = jnp.sum((rh != 0.0).astype(jnp.float32))
    cond = lcnt > rcnt

    hand = jnp.where(cond, lh, rh)
    fs = jnp.sum(hand, axis=2)
    mask = (fs != 0.0).astype(jnp.float32)

    jj = jax.lax.broadcasted_iota(jnp.int32, (_C, _C), 0)
    kk = jax.lax.broadcasted_iota(jnp.int32, (_C, _C), 1)
    tri = (jj <= kk).astype(jnp.float32)
    crow = jnp.dot(mask, tri, precision=_DF)
    tot = crow[:, _C - 1:_C]
    aa = jax.lax.broadcasted_iota(jnp.int32, (_R, _R), 0)
    bb = jax.lax.broadcasted_iota(jnp.int32, (_R, _R), 1)
    lstrict = (bb < aa).astype(jnp.float32)
    pre = jnp.dot(lstrict, tot, precision=_DF)
    kinc = crow + pre
    nkept = pre[_R - 1:_R, 0:1] + tot[_R - 1:_R, 0:1]

    gr = jax.lax.broadcasted_iota(jnp.int32, (_R, _C), 0)
    gj = jax.lax.broadcasted_iota(jnp.int32, (_R, _C), 1)
    gidx = (gr * _C + gj).astype(jnp.float32)
    dinc = (gidx + 1.0) - kinc
    pos = jnp.where(mask > 0, kinc - 1.0, nkept + dinc - 1.0)

    p_iota = jax.lax.broadcasted_iota(jnp.int32, (_R, _C, _NS), 2)
    pos3 = pos.astype(jnp.int32)[:, :, None]
    oh = (pos3 == p_iota).astype(jnp.float32).reshape(_F, _NS)
    g = jax.lax.dot_general(oh, x, (((0,), (0,)), ((), ())), precision=_HI)

    h = jnp.where(cond, g[:, 120:183], g[:, 282:345])
    xf = jnp.concatenate([h, g[:, 183:258], g[:, 0:120]], axis=1)
    lane = jax.lax.broadcasted_iota(jnp.int32, (_NS, _XF), 1)
    neg = jnp.logical_and(cond, lane % 3 == 0)
    xf = jnp.where(neg, -xf, xf)

    y = jnp.dot(xf, fm_ref[...], precision=_HI)
    coords = y[:, 0:153]
    dd = y[:, 153:]

    sh = (kk == jj + 1).astype(jnp.float32)
    coords_next = jnp.dot(sh, coords, precision=_HI)
    s_iota = jax.lax.broadcasted_iota(jnp.int32, (_NS, 1), 0).astype(jnp.float32)
    nk = (s_iota < (nkept - 1.0)).astype(jnp.float32)
    dcoords = (coords - coords_next) * nk

    hd = jnp.sqrt(dd[:, 0:210] ** 2 + dd[:, 210:420] ** 2 + dd[:, 420:630] ** 2)
    pd = jnp.sqrt(dd[:, 630:930] ** 2 + dd[:, 930:1230] ** 2)
    od = jnp.sqrt(dd[:, 1230:1420] ** 2 + dd[:, 1420:1610] ** 2)
    idd = jnp.sqrt(dd[:, 1610:1800] ** 2 + dd[:, 1800:1990] ** 2)

    out_ref[:, 0:153] = coords
    out_ref[:, 153:306] = dcoords
    out_ref[:, 306:516] = hd
    out_ref[:, 516:816] = pd
    out_ref[:, 816:1006] = od
    out_ref[:, 1006:1196] = idd


def kernel(x):
    x2 = x.reshape(_F, _L)
    fm = jnp.asarray(_FM_NP)
    out = pl.pallas_call(
        _fgen_kernel,
        out_shape=jax.ShapeDtypeStruct((_NS, 1196), jnp.float32),
    )(x2, fm)
    return out[:100].reshape(1, 100, 1196)


# speedup vs baseline: 14.5845x; 1.0700x over previous
import numpy as np
import jax
import jax.numpy as jnp
from jax.experimental import pallas as pl

_F = 2048
_L = 345
_R = 16
_C = 128
_NS = 128
_XF = 258
_HI = jax.lax.Precision.HIGHEST
_DF = jax.lax.Precision.DEFAULT


def _build_consts():
    cols = []
    sel = np.zeros((_XF, 153), np.float32)
    for i in range(63):
        sel[i, i] = 1.0
    for p in range(25):
        for cc in range(2):
            sel[63 + 3 * p + cc, 63 + 2 * p + cc] = 1.0
    for p in range(20):
        for cc in range(2):
            sel[138 + 3 * p + cc, 113 + 2 * p + cc] = 1.0
    cols.append(sel)

    def add_region(npts, ncoord, base):
        r, c = np.triu_indices(npts, k=1)
        for cc in range(ncoord):
            m = np.zeros((_XF, len(r)), np.float32)
            m[base + 3 * r + cc, np.arange(len(r))] = 1.0
            m[base + 3 * c + cc, np.arange(len(r))] -= 1.0
            cols.append(m)

    add_region(21, 3, 0)
    add_region(25, 2, 63)
    add_region(20, 2, 138)
    add_region(20, 2, 198)
    return np.concatenate(cols, axis=1)


_FM_NP = _build_consts()


def _fgen_kernel(x_ref, fm_ref, out_ref):
    x = x_ref[...]
    x = jnp.where(jnp.isnan(x), jnp.float32(0.0), x)
    xr = x.reshape(_R, _C, _L)
    lh = xr[:, :, 120:183]
    rh = xr[:, :, 282:345]
    lcnt = jnp.sum((lh != 0.0).astype(jnp.float32))
    rcnt = jnp.sum((rh != 0.0).astype(jnp.float32))
    cond = lcnt > rcnt

    hand = jnp.where(cond, lh, rh)
    fs = jnp.sum(hand, axis=2)
    mask = (fs != 0.0).astype(jnp.float32)

    jj = jax.lax.broadcasted_iota(jnp.int32, (_C, _C), 0)
    kk = jax.lax.broadcasted_iota(jnp.int32, (_C, _C), 1)
    tri = (jj <= kk).astype(jnp.bfloat16)
    crow = jnp.dot(mask.astype(jnp.bfloat16), tri,
                   preferred_element_type=jnp.float32)
    tot = crow[:, _C - 1:_C]
    aa = jax.lax.broadcasted_iota(jnp.int32, (_R, _R), 0)
    bb = jax.lax.broadcasted_iota(jnp.int32, (_R, _R), 1)
    lstrict = (bb < aa).astype(jnp.bfloat16)
    pre = jnp.dot(lstrict, tot.astype(jnp.bfloat16),
                  preferred_element_type=jnp.float32)
    kinc = crow + pre
    nkept = pre[_R - 1:_R, 0:1] + tot[_R - 1:_R, 0:1]

    gr = jax.lax.broadcasted_iota(jnp.int32, (_R, _C), 0)
    gj = jax.lax.broadcasted_iota(jnp.int32, (_R, _C), 1)
    gidx = (gr * _C + gj).astype(jnp.float32)
    dinc = (gidx + 1.0) - kinc
    pos = jnp.where(mask > 0, kinc - 1.0, nkept + dinc - 1.0)

    p_iota = jax.lax.broadcasted_iota(jnp.int32, (_R, _C, _NS), 2)
    pos3 = pos.astype(jnp.int32)[:, :, None]
    oh = (pos3 == p_iota).astype(jnp.bfloat16).reshape(_F, _NS)
    x_hi = x.astype(jnp.bfloat16)
    x_lo = (x - x_hi.astype(jnp.float32)).astype(jnp.bfloat16)
    dn = (((0,), (0,)), ((), ()))
    g = (jax.lax.dot_general(oh, x_hi, dn, preferred_element_type=jnp.float32)
         + jax.lax.dot_general(oh, x_lo, dn, preferred_element_type=jnp.float32))

    h = jnp.where(cond, g[:, 120:183], g[:, 282:345])
    xf = jnp.concatenate([h, g[:, 183:258], g[:, 0:120]], axis=1)
    lane = jax.lax.broadcasted_iota(jnp.int32, (_NS, _XF), 1)
    neg = jnp.logical_and(cond, lane % 3 == 0)
    xf = jnp.where(neg, -xf, xf)

    fm = fm_ref[...]
    xf_hi = xf.astype(jnp.bfloat16)
    xf_lo = (xf - xf_hi.astype(jnp.float32)).astype(jnp.bfloat16)
    y = (jnp.dot(xf_hi, fm, preferred_element_type=jnp.float32)
         + jnp.dot(xf_lo, fm, preferred_element_type=jnp.float32))
    coords = y[:, 0:153]
    dd = y[:, 153:]

    sh = (kk == jj + 1).astype(jnp.float32)
    coords_next = jnp.dot(sh, coords, precision=_HI)
    s_iota = jax.lax.broadcasted_iota(jnp.int32, (_NS, 1), 0).astype(jnp.float32)
    nk = (s_iota < (nkept - 1.0)).astype(jnp.float32)
    dcoords = (coords - coords_next) * nk

    hd = jnp.sqrt(dd[:, 0:210] ** 2 + dd[:, 210:420] ** 2 + dd[:, 420:630] ** 2)
    pd = jnp.sqrt(dd[:, 630:930] ** 2 + dd[:, 930:1230] ** 2)
    od = jnp.sqrt(dd[:, 1230:1420] ** 2 + dd[:, 1420:1610] ** 2)
    idd = jnp.sqrt(dd[:, 1610:1800] ** 2 + dd[:, 1800:1990] ** 2)

    out_ref[:, 0:153] = coords
    out_ref[:, 153:306] = dcoords
    out_ref[:, 306:516] = hd
    out_ref[:, 516:816] = pd
    out_ref[:, 816:1006] = od
    out_ref[:, 1006:1196] = idd


def kernel(x):
    x2 = x.reshape(_F, _L)
    fm = jnp.asarray(_FM_NP, dtype=jnp.bfloat16)
    out = pl.pallas_call(
        _fgen_kernel,
        out_shape=jax.ShapeDtypeStruct((_NS, 1196), jnp.float32),
    )(x2, fm)
    return out[:100].reshape(1, 100, 1196)
